# baseline (speedup 1.0000x reference)
import functools

import jax
import jax.numpy as jnp
from jax import lax
from jax.experimental import pallas as pl
from jax.experimental.pallas import tpu as pltpu, tpu_sc as plsc

QN = 4000
CN = 1000
N = QN + CN
EMB = 128
HID = 128
HEADS = 4
RSTEPS = 9

NTILES = 32
_MESH = plsc.VectorSubcoreMesh(core_axis_name="c", subcore_axis_name="s")
_SC_PARAMS = pltpu.CompilerParams(needs_layout_passes=False)


def _make_sc_gat(NN, NT, H, D, EE, EEp, CH):
    nd = D // NTILES
    n_chunks = EEp // CH
    grp = CH // 16
    tiles_per_head = NTILES // H

    @functools.partial(
        pl.kernel, mesh=_MESH, compiler_params=_SC_PARAMS,
        out_type=(jax.ShapeDtypeStruct((D * NT,), jnp.float32),
                  jax.ShapeDtypeStruct((NTILES * NT,), jnp.float32)),
        scratch_types=[
            pltpu.VMEM((CH,), jnp.int32),
            pltpu.VMEM((CH,), jnp.int32),
            pltpu.VMEM((NT,), jnp.float32),
            pltpu.VMEM((NT,), jnp.float32),
            pltpu.VMEM((NT,), jnp.float32),
            pltpu.VMEM((nd * NT,), jnp.float32),
            pltpu.VMEM((nd * NT,), jnp.float32),
        ],
    )
    def gat_edges(asrc_hbm, adst_hbm, h_hbm, src_hbm, dst_hbm,
                  out_hbm, den_hbm,
                  src_c, dst_c, asrc_v, adst_v, den_v, h_v, out_v):
        wid = lax.axis_index("s") * 2 + lax.axis_index("c")
        head = wid // tiles_per_head
        pltpu.sync_copy(asrc_hbm.at[pl.ds(head * NT, NT)], asrc_v)
        pltpu.sync_copy(adst_hbm.at[pl.ds(head * NT, NT)], adst_v)
        pltpu.sync_copy(h_hbm.at[pl.ds(wid * (nd * NT), nd * NT)], h_v)

        zero16 = jnp.zeros((16,), jnp.float32)

        def zloop(i, carry):
            den_v[pl.ds(i * 16, 16)] = zero16
            return carry

        lax.fori_loop(0, NT // 16, zloop, 0)

        def zloop2(i, carry):
            out_v[pl.ds(i * 16, 16)] = zero16
            return carry

        lax.fori_loop(0, nd * NT // 16, zloop2, 0)

        lane = lax.iota(jnp.int32, 16)

        def chunk_a(ci, carry):
            pltpu.sync_copy(src_hbm.at[pl.ds(ci * CH, CH)], src_c)
            pltpu.sync_copy(dst_hbm.at[pl.ds(ci * CH, CH)], dst_c)

            @plsc.parallel_loop(0, grp, unroll=8)
            def grp_a(g):
                s16 = src_c[pl.ds(g * 16, 16)]
                d16 = dst_c[pl.ds(g * 16, 16)]
                a = (plsc.load_gather(asrc_v, [s16])
                     + plsc.load_gather(adst_v, [d16]))
                a = jnp.where(a > 0, a, a * 0.2)
                ex = jnp.exp(a)
                mask = (ci * CH + g * 16 + lane) < EE
                plsc.addupdate_scatter(den_v, [d16], ex, mask=mask)

            return carry

        lax.fori_loop(0, n_chunks, chunk_a, 0)

        def chunk_b(ci, carry):
            pltpu.sync_copy(src_hbm.at[pl.ds(ci * CH, CH)], src_c)
            pltpu.sync_copy(dst_hbm.at[pl.ds(ci * CH, CH)], dst_c)

            @plsc.parallel_loop(0, grp, unroll=4)
            def grp_b(g):
                s16 = src_c[pl.ds(g * 16, 16)]
                d16 = dst_c[pl.ds(g * 16, 16)]
                a = (plsc.load_gather(asrc_v, [s16])
                     + plsc.load_gather(adst_v, [d16]))
                a = jnp.where(a > 0, a, a * 0.2)
                ex = jnp.exp(a)
                al = ex / plsc.load_gather(den_v, [d16])
                mask = (ci * CH + g * 16 + lane) < EE
                for d in range(nd):
                    hv = plsc.load_gather(h_v, [s16 + d * NT])
                    plsc.addupdate_scatter(out_v, [d16 + d * NT],
                                           hv * al, mask=mask)

            return carry

        lax.fori_loop(0, n_chunks, chunk_b, 0)

        pltpu.sync_copy(out_v, out_hbm.at[pl.ds(wid * (nd * NT), nd * NT)])
        pltpu.sync_copy(den_v, den_hbm.at[pl.ds(wid * NT, NT)])

    return gat_edges


def _pad_nodes(x, NT):
    H, NN = x.shape
    return jnp.pad(x, ((0, 0), (0, NT - NN))).reshape(-1)


def _sc_gat_layer(h_nodes, a_src, a_dst, src_e, dst_e, NN, NT, H, EE, EEp, CH):
    D = h_nodes.shape[1]
    h_flat = _pad_nodes(h_nodes.T, NT)
    asrc_flat = _pad_nodes(a_src, NT)
    adst_flat = _pad_nodes(a_dst, NT)
    fn = _make_sc_gat(NN, NT, H, D, EE, EEp, CH)
    out_flat, den_flat = fn(asrc_flat, adst_flat, h_flat, src_e, dst_e)
    out = out_flat.reshape(D, NT)[:, :NN].T
    den_rows = den_flat.reshape(NTILES, NT)[:: NTILES // H, :NN]
    return out, den_rows


_A0R = 125008
_A0CH = 2048


def _make_sc_attn0(EE, EEp):
    n_chunks = EEp // _A0CH
    grp = _A0CH // 16

    @functools.partial(
        pl.kernel, mesh=_MESH, compiler_params=_SC_PARAMS,
        out_type=jax.ShapeDtypeStruct((NTILES * _A0R,), jnp.float32),
        scratch_types=[
            pltpu.VMEM((_A0CH,), jnp.int32),
            pltpu.VMEM((_A0CH,), jnp.float32),
            pltpu.VMEM((_A0R,), jnp.float32),
        ],
    )
    def attn0_scatter(fi_hbm, val_hbm, z_hbm, out_hbm, fi_c, val_c, tab_v):
        wid = lax.axis_index("s") * 2 + lax.axis_index("c")
        lo = wid * _A0R
        pltpu.sync_copy(z_hbm, tab_v)
        lane = lax.iota(jnp.int32, 16)

        def chunk(ci, carry):
            pltpu.sync_copy(fi_hbm.at[pl.ds(ci * _A0CH, _A0CH)], fi_c)
            pltpu.sync_copy(val_hbm.at[pl.ds(ci * _A0CH, _A0CH)], val_c)

            @plsc.parallel_loop(0, grp, unroll=8)
            def grp_f(g):
                f16 = fi_c[pl.ds(g * 16, 16)]
                v16 = val_c[pl.ds(g * 16, 16)]
                mask = ((f16 >= lo) & (f16 < lo + _A0R)
                        & ((ci * _A0CH + g * 16 + lane) < EE))
                loc = jnp.where(mask, f16 - lo, 0)
                plsc.addupdate_scatter(tab_v, [loc], v16, mask=mask)

            return carry

        lax.fori_loop(0, n_chunks, chunk, 0)
        pltpu.sync_copy(tab_v, out_hbm.at[pl.ds(lo, _A0R)])

    return attn0_scatter


def _route_lstm_body(rt_ref, idx_ref, wcat_ref, b_ref, out_ref):
    blk = rt_ref.shape[0]
    x = rt_ref[...]
    wcat = wcat_ref[...]
    b = b_ref[...]
    idx = idx_ref[...]
    h = jnp.zeros((blk, HID), jnp.float32)
    c = jnp.zeros((blk, HID), jnp.float32)
    out = jnp.zeros((blk, HID), jnp.float32)
    for t in range(RSTEPS):
        xt = x[:, t, :]
        g = jnp.concatenate([xt, h], axis=1) @ wcat + b
        i = jax.nn.sigmoid(g[:, :HID])
        f = jax.nn.sigmoid(g[:, HID:2 * HID])
        gg = jnp.tanh(g[:, 2 * HID:3 * HID])
        o = jax.nn.sigmoid(g[:, 3 * HID:])
        c = f * c + i * gg
        h = o * jnp.tanh(c)
        out = jnp.where(idx == t, h, out)
    out_ref[...] = out


def _route_lstm(rt, idx, wcat, b, blk):
    n = rt.shape[0]
    grid = n // blk
    idx2 = idx.reshape(n, 1)
    return pl.pallas_call(
        _route_lstm_body,
        grid=(grid,),
        in_specs=[
            pl.BlockSpec((blk, RSTEPS, EMB), lambda i: (i, 0, 0)),
            pl.BlockSpec((blk, 1), lambda i: (i, 0)),
            pl.BlockSpec((EMB + HID, 4 * HID), lambda i: (0, 0)),
            pl.BlockSpec((1, 4 * HID), lambda i: (0, 0)),
        ],
        out_specs=pl.BlockSpec((blk, HID), lambda i: (i, 0)),
        out_shape=jax.ShapeDtypeStruct((n, HID), jnp.float32),
    )(rt, idx2, wcat, b)


def _kt_body(xcat_ref, fc0w_ref, fc0b_ref, wih_ref, whh_ref, bk_ref,
             fc1w_ref, fc1b_ref, ind_ref, state_ref, pred_ref,
             gi_ref, hs_ref):
    L, B, F = xcat_ref.shape
    x0 = jax.nn.relu(
        xcat_ref[...].reshape(L * B, F) @ fc0w_ref[...] + fc0b_ref[...])
    gi_ref[...] = (x0 @ wih_ref[...] + bk_ref[...]).reshape(L, B, 4 * HID)

    whh = whh_ref[...]

    def step(t, hc):
        h, c = hc
        g = gi_ref[t] + h @ whh
        i = jax.nn.sigmoid(g[:, :HID])
        f = jax.nn.sigmoid(g[:, HID:2 * HID])
        gg = jnp.tanh(g[:, 2 * HID:3 * HID])
        o = jax.nn.sigmoid(g[:, 3 * HID:])
        c = f * c + i * gg
        h = o * jnp.tanh(c)
        hs_ref[t] = h
        return (h, c)

    h0 = jnp.zeros((B, HID), jnp.float32)
    c0 = jnp.zeros((B, HID), jnp.float32)
    jax.lax.fori_loop(0, L, step, (h0, c0))

    state = jax.nn.sigmoid(
        hs_ref[...].reshape(L * B, HID) @ fc1w_ref[...] + fc1b_ref[...])
    state = state.reshape(L, B, CN)
    state_ref[...] = state

    ind = ind_ref[...]
    whole = jnp.sum(ind, axis=-1)
    whole = jnp.where(whole > 0.0, whole, 1.0)
    pred_ref[...] = jnp.sum(state[:L - 1] * ind, axis=-1) / whole


def _kt_chain(xcat_lm, fc0w, fc0b, wih, whh, bk, fc1w, fc1b, ind_lm):
    L, B, F = xcat_lm.shape
    out_shape = (
        jax.ShapeDtypeStruct((L, B, CN), jnp.float32),
        jax.ShapeDtypeStruct((L - 1, B), jnp.float32),
    )
    return pl.pallas_call(
        _kt_body,
        out_shape=out_shape,
        scratch_shapes=[
            pltpu.VMEM((L, B, 4 * HID), jnp.float32),
            pltpu.VMEM((L, B, HID), jnp.float32),
        ],
    )(xcat_lm, fc0w, fc0b, wih, whh, bk, fc1w, fc1b, ind_lm)


def _pad_edges(src, dst, num_nodes, EEp):
    loops = jnp.arange(num_nodes, dtype=src.dtype)
    s = jnp.concatenate([src, loops])
    d = jnp.concatenate([dst, loops])
    pad = EEp - s.shape[0]
    return jnp.pad(s, (0, pad)), jnp.pad(d, (0, pad))


def kernel(students, questions, features, features_len, routes, routes_len,
           answers, whole_edge_index, whole_edge_attr, edge_index, edge_attr,
           lin0_w, att_src0, att_dst0, b0, lin1_w, att_src1, att_dst1, b1,
           lin2_w, att_src2, att_dst2, b2, result_emb, Wih_r, Whh_r, bih_r,
           bhh_r, fc0_w, fc0_b, Wih_k, Whh_k, bih_k, bhh_k, fc1_w, fc1_b):
    B, L = questions.shape

    h0 = lin0_w
    a_src0 = (h0 @ att_src0[0])[None, :]
    a_dst0 = (h0 @ att_dst0[0])[None, :]
    src0 = whole_edge_index[0]
    dst0 = whole_edge_index[1]
    E0 = src0.shape[0]
    EE0, EEp0 = E0 + N, 57344
    s0e, d0e = _pad_edges(src0, dst0, N, EEp0)
    out0, den0 = _sc_gat_layer(h0, a_src0, a_dst0, s0e, d0e,
                               NN=N, NT=5008, H=1, EE=EE0, EEp=EEp0, CH=8192)
    W = out0 + b0

    al0 = jax.nn.leaky_relu(a_src0[0][src0] + a_dst0[0][dst0], 0.2)
    a0 = jnp.exp(al0) / den0[0][dst0]
    mask0 = (src0 < QN) & (dst0 >= QN)
    fi = jnp.where(mask0, src0 * CN + (dst0 - QN), 0)
    val = jnp.where(mask0, a0, 0.0)
    EEp_a = 51200
    fi = jnp.pad(fi, (0, EEp_a - E0))
    val = jnp.pad(val, (0, EEp_a - E0))
    zrow = jnp.zeros((_A0R,), jnp.float32)
    attn0_flat = _make_sc_attn0(E0, EEp_a)(fi, val, zrow)
    attn0 = attn0_flat[:QN * CN].reshape(QN, CN)

    X = W[QN:]
    s1 = edge_index[0]
    d1 = edge_index[1]
    EE1, EEp1 = s1.shape[0] + CN, 24576
    s1e, d1e = _pad_edges(s1, d1, CN, EEp1)

    h1 = X @ lin1_w
    h1h = h1.reshape(CN, HEADS, EMB)
    as1 = (h1h * att_src1).sum(-1).T
    ad1 = (h1h * att_dst1).sum(-1).T
    out1, _ = _sc_gat_layer(h1, as1, ad1, s1e, d1e,
                            NN=CN, NT=1008, H=HEADS, EE=EE1, EEp=EEp1,
                            CH=8192)
    X = jax.nn.relu(out1 + b1)

    h2 = X @ lin2_w
    h2h = h2.reshape(CN, HEADS, EMB)
    as2 = (h2h * att_src2).sum(-1).T
    ad2 = (h2h * att_dst2).sum(-1).T
    out2, _ = _sc_gat_layer(h2, as2, ad2, s1e, d1e,
                            NN=CN, NT=1008, H=HEADS, EE=EE1, EEp=EEp1,
                            CH=8192)
    X2 = out2.reshape(CN, HEADS, EMB).mean(axis=1) + b2

    routes9 = routes[..., :RSTEPS].reshape(-1)
    rt = X2[routes9].reshape(B * L, RSTEPS, EMB)
    idx = jnp.maximum(routes_len.reshape(-1) - 1, 0).astype(jnp.int32)
    wcat_r = jnp.concatenate([Wih_r.T, Whh_r.T], axis=0)
    br = (bih_r + bhh_r)[None, :]
    xt = _route_lstm(rt, idx, wcat_r, br, blk=800)
    xt = xt.reshape(B, L, HID)

    qt = W[questions]
    r = result_emb[answers]
    xcat = jnp.concatenate([qt, xt, r], axis=-1)
    xcat_lm = jnp.swapaxes(xcat, 0, 1)
    ind = attn0[questions[:, 1:]]
    ind_lm = jnp.swapaxes(ind, 0, 1)

    state_lm, pred_lm = _kt_chain(
        xcat_lm, fc0_w, fc0_b[None, :], Wih_k.T, Whh_k.T,
        (bih_k + bhh_k)[None, :], fc1_w, fc1_b[None, :], ind_lm)

    state = jnp.swapaxes(state_lm, 0, 1)
    pred = jnp.swapaxes(pred_lm, 0, 1)
    return (attn0, state, pred)


# speedup vs baseline: 1.0410x; 1.0410x over previous
import functools

import jax
import jax.numpy as jnp
from jax import lax
from jax.experimental import pallas as pl
from jax.experimental.pallas import tpu as pltpu, tpu_sc as plsc

QN = 4000
CN = 1000
N = QN + CN
EMB = 128
HID = 128
HEADS = 4
RSTEPS = 9

NTILES = 32
_MESH = plsc.VectorSubcoreMesh(core_axis_name="c", subcore_axis_name="s")
_SC_PARAMS = pltpu.CompilerParams(needs_layout_passes=False)


def _make_sc_gat(NN, NT, H, D, EE, EEp, CH):
    nd = D // NTILES
    n_chunks = EEp // CH
    grp = CH // 16
    tiles_per_head = NTILES // H

    @functools.partial(
        pl.kernel, mesh=_MESH, compiler_params=_SC_PARAMS,
        out_type=(jax.ShapeDtypeStruct((D * NT,), jnp.float32),
                  jax.ShapeDtypeStruct((NTILES * NT,), jnp.float32)),
        scratch_types=[
            pltpu.VMEM((CH,), jnp.int32),
            pltpu.VMEM((CH,), jnp.int32),
            pltpu.VMEM((NT,), jnp.float32),
            pltpu.VMEM((NT,), jnp.float32),
            pltpu.VMEM((NT,), jnp.float32),
            pltpu.VMEM((nd * NT,), jnp.float32),
            pltpu.VMEM((nd * NT,), jnp.float32),
        ],
    )
    def gat_edges(asrc_hbm, adst_hbm, h_hbm, src_hbm, dst_hbm,
                  out_hbm, den_hbm,
                  src_c, dst_c, asrc_v, adst_v, den_v, h_v, out_v):
        wid = lax.axis_index("s") * 2 + lax.axis_index("c")
        head = wid // tiles_per_head
        pltpu.sync_copy(asrc_hbm.at[pl.ds(head * NT, NT)], asrc_v)
        pltpu.sync_copy(adst_hbm.at[pl.ds(head * NT, NT)], adst_v)
        pltpu.sync_copy(h_hbm.at[pl.ds(wid * (nd * NT), nd * NT)], h_v)

        zero16 = jnp.zeros((16,), jnp.float32)

        def zloop(i, carry):
            den_v[pl.ds(i * 16, 16)] = zero16
            return carry

        lax.fori_loop(0, NT // 16, zloop, 0)

        def zloop2(i, carry):
            out_v[pl.ds(i * 16, 16)] = zero16
            return carry

        lax.fori_loop(0, nd * NT // 16, zloop2, 0)

        lane = lax.iota(jnp.int32, 16)

        def chunk_a(ci, carry):
            pltpu.sync_copy(src_hbm.at[pl.ds(ci * CH, CH)], src_c)
            pltpu.sync_copy(dst_hbm.at[pl.ds(ci * CH, CH)], dst_c)

            @plsc.parallel_loop(0, grp, unroll=4)
            def grp_a(g):
                s16 = src_c[pl.ds(g * 16, 16)]
                d16 = dst_c[pl.ds(g * 16, 16)]
                a = (plsc.load_gather(asrc_v, [s16])
                     + plsc.load_gather(adst_v, [d16]))
                a = jnp.where(a > 0, a, a * 0.2)
                ex = jnp.exp(a)
                mask = (ci * CH + g * 16 + lane) < EE
                plsc.addupdate_scatter(den_v, [d16], ex, mask=mask)

            return carry

        lax.fori_loop(0, n_chunks, chunk_a, 0)

        def chunk_b(ci, carry):
            pltpu.sync_copy(src_hbm.at[pl.ds(ci * CH, CH)], src_c)
            pltpu.sync_copy(dst_hbm.at[pl.ds(ci * CH, CH)], dst_c)

            @plsc.parallel_loop(0, grp, unroll=2)
            def grp_b(g):
                s16 = src_c[pl.ds(g * 16, 16)]
                d16 = dst_c[pl.ds(g * 16, 16)]
                a = (plsc.load_gather(asrc_v, [s16])
                     + plsc.load_gather(adst_v, [d16]))
                a = jnp.where(a > 0, a, a * 0.2)
                ex = jnp.exp(a)
                al = ex / plsc.load_gather(den_v, [d16])
                mask = (ci * CH + g * 16 + lane) < EE
                for d in range(nd):
                    hv = plsc.load_gather(h_v, [s16 + d * NT])
                    plsc.addupdate_scatter(out_v, [d16 + d * NT],
                                           hv * al, mask=mask)

            return carry

        lax.fori_loop(0, n_chunks, chunk_b, 0)

        pltpu.sync_copy(out_v, out_hbm.at[pl.ds(wid * (nd * NT), nd * NT)])
        pltpu.sync_copy(den_v, den_hbm.at[pl.ds(wid * NT, NT)])

    return gat_edges


def _pad_nodes(x, NT):
    H, NN = x.shape
    return jnp.pad(x, ((0, 0), (0, NT - NN))).reshape(-1)


def _sc_gat_layer(h_nodes, a_src, a_dst, src_e, dst_e, NN, NT, H, EE, EEp, CH):
    D = h_nodes.shape[1]
    h_flat = _pad_nodes(h_nodes.T, NT)
    asrc_flat = _pad_nodes(a_src, NT)
    adst_flat = _pad_nodes(a_dst, NT)
    fn = _make_sc_gat(NN, NT, H, D, EE, EEp, CH)
    out_flat, den_flat = fn(asrc_flat, adst_flat, h_flat, src_e, dst_e)
    out = out_flat.reshape(D, NT)[:, :NN].T
    den_rows = den_flat.reshape(NTILES, NT)[:: NTILES // H, :NN]
    return out, den_rows


_A0R = 125008
_A0CH = 2048


def _make_sc_attn0(EE, EEp):
    n_chunks = EEp // _A0CH
    grp = _A0CH // 16

    @functools.partial(
        pl.kernel, mesh=_MESH, compiler_params=_SC_PARAMS,
        out_type=jax.ShapeDtypeStruct((NTILES * _A0R,), jnp.float32),
        scratch_types=[
            pltpu.VMEM((_A0CH,), jnp.int32),
            pltpu.VMEM((_A0CH,), jnp.float32),
            pltpu.VMEM((_A0R,), jnp.float32),
        ],
    )
    def attn0_scatter(fi_hbm, val_hbm, z_hbm, out_hbm, fi_c, val_c, tab_v):
        wid = lax.axis_index("s") * 2 + lax.axis_index("c")
        lo = wid * _A0R
        pltpu.sync_copy(z_hbm, tab_v)
        lane = lax.iota(jnp.int32, 16)

        def chunk(ci, carry):
            pltpu.sync_copy(fi_hbm.at[pl.ds(ci * _A0CH, _A0CH)], fi_c)
            pltpu.sync_copy(val_hbm.at[pl.ds(ci * _A0CH, _A0CH)], val_c)

            @plsc.parallel_loop(0, grp, unroll=4)
            def grp_f(g):
                f16 = fi_c[pl.ds(g * 16, 16)]
                v16 = val_c[pl.ds(g * 16, 16)]
                mask = ((f16 >= lo) & (f16 < lo + _A0R)
                        & ((ci * _A0CH + g * 16 + lane) < EE))
                loc = jnp.where(mask, f16 - lo, 0)
                plsc.addupdate_scatter(tab_v, [loc], v16, mask=mask)

            return carry

        lax.fori_loop(0, n_chunks, chunk, 0)
        pltpu.sync_copy(tab_v, out_hbm.at[pl.ds(lo, _A0R)])

    return attn0_scatter


def _route_lstm_body(rt_ref, idx_ref, wcat_ref, b_ref, out_ref):
    blk = rt_ref.shape[0]
    x = rt_ref[...]
    wcat = wcat_ref[...]
    b = b_ref[...]
    idx = idx_ref[...]
    h = jnp.zeros((blk, HID), jnp.float32)
    c = jnp.zeros((blk, HID), jnp.float32)
    out = jnp.zeros((blk, HID), jnp.float32)
    for t in range(RSTEPS):
        xt = x[:, t, :]
        g = jnp.concatenate([xt, h], axis=1) @ wcat + b
        i = jax.nn.sigmoid(g[:, :HID])
        f = jax.nn.sigmoid(g[:, HID:2 * HID])
        gg = jnp.tanh(g[:, 2 * HID:3 * HID])
        o = jax.nn.sigmoid(g[:, 3 * HID:])
        c = f * c + i * gg
        h = o * jnp.tanh(c)
        out = jnp.where(idx == t, h, out)
    out_ref[...] = out


def _route_lstm(rt, idx, wcat, b, blk):
    n = rt.shape[0]
    grid = n // blk
    idx2 = idx.reshape(n, 1)
    return pl.pallas_call(
        _route_lstm_body,
        grid=(grid,),
        in_specs=[
            pl.BlockSpec((blk, RSTEPS, EMB), lambda i: (i, 0, 0)),
            pl.BlockSpec((blk, 1), lambda i: (i, 0)),
            pl.BlockSpec((EMB + HID, 4 * HID), lambda i: (0, 0)),
            pl.BlockSpec((1, 4 * HID), lambda i: (0, 0)),
        ],
        out_specs=pl.BlockSpec((blk, HID), lambda i: (i, 0)),
        out_shape=jax.ShapeDtypeStruct((n, HID), jnp.float32),
    )(rt, idx2, wcat, b)


def _kt_body(xcat_ref, fc0w_ref, fc0b_ref, wih_ref, whh_ref, bk_ref,
             fc1w_ref, fc1b_ref, ind_ref, state_ref, pred_ref,
             gi_ref, hs_ref):
    L, B, F = xcat_ref.shape
    x0 = jax.nn.relu(
        xcat_ref[...].reshape(L * B, F) @ fc0w_ref[...] + fc0b_ref[...])
    gi_ref[...] = (x0 @ wih_ref[...] + bk_ref[...]).reshape(L, B, 4 * HID)

    whh = whh_ref[...]

    def step(t, hc):
        h, c = hc
        g = gi_ref[t] + h @ whh
        i = jax.nn.sigmoid(g[:, :HID])
        f = jax.nn.sigmoid(g[:, HID:2 * HID])
        gg = jnp.tanh(g[:, 2 * HID:3 * HID])
        o = jax.nn.sigmoid(g[:, 3 * HID:])
        c = f * c + i * gg
        h = o * jnp.tanh(c)
        hs_ref[t] = h
        return (h, c)

    h0 = jnp.zeros((B, HID), jnp.float32)
    c0 = jnp.zeros((B, HID), jnp.float32)
    jax.lax.fori_loop(0, L, step, (h0, c0))

    state = jax.nn.sigmoid(
        hs_ref[...].reshape(L * B, HID) @ fc1w_ref[...] + fc1b_ref[...])
    state = state.reshape(L, B, CN)
    state_ref[...] = state

    ind = ind_ref[...]
    whole = jnp.sum(ind, axis=-1)
    whole = jnp.where(whole > 0.0, whole, 1.0)
    pred_ref[...] = jnp.sum(state[:L - 1] * ind, axis=-1) / whole


def _kt_chain(xcat_lm, fc0w, fc0b, wih, whh, bk, fc1w, fc1b, ind_lm):
    L, B, F = xcat_lm.shape
    out_shape = (
        jax.ShapeDtypeStruct((L, B, CN), jnp.float32),
        jax.ShapeDtypeStruct((L - 1, B), jnp.float32),
    )
    return pl.pallas_call(
        _kt_body,
        out_shape=out_shape,
        scratch_shapes=[
            pltpu.VMEM((L, B, 4 * HID), jnp.float32),
            pltpu.VMEM((L, B, HID), jnp.float32),
        ],
    )(xcat_lm, fc0w, fc0b, wih, whh, bk, fc1w, fc1b, ind_lm)


def _pad_edges(src, dst, num_nodes, EEp):
    loops = jnp.arange(num_nodes, dtype=src.dtype)
    s = jnp.concatenate([src, loops])
    d = jnp.concatenate([dst, loops])
    pad = EEp - s.shape[0]
    return jnp.pad(s, (0, pad)), jnp.pad(d, (0, pad))


def kernel(students, questions, features, features_len, routes, routes_len,
           answers, whole_edge_index, whole_edge_attr, edge_index, edge_attr,
           lin0_w, att_src0, att_dst0, b0, lin1_w, att_src1, att_dst1, b1,
           lin2_w, att_src2, att_dst2, b2, result_emb, Wih_r, Whh_r, bih_r,
           bhh_r, fc0_w, fc0_b, Wih_k, Whh_k, bih_k, bhh_k, fc1_w, fc1_b):
    B, L = questions.shape

    h0 = lin0_w
    a_src0 = (h0 @ att_src0[0])[None, :]
    a_dst0 = (h0 @ att_dst0[0])[None, :]
    src0 = whole_edge_index[0]
    dst0 = whole_edge_index[1]
    E0 = src0.shape[0]
    EE0, EEp0 = E0 + N, 57344
    s0e, d0e = _pad_edges(src0, dst0, N, EEp0)
    out0, den0 = _sc_gat_layer(h0, a_src0, a_dst0, s0e, d0e,
                               NN=N, NT=5008, H=1, EE=EE0, EEp=EEp0, CH=8192)
    W = out0 + b0

    al0 = jax.nn.leaky_relu(a_src0[0][src0] + a_dst0[0][dst0], 0.2)
    a0 = jnp.exp(al0) / den0[0][dst0]
    mask0 = (src0 < QN) & (dst0 >= QN)
    fi = jnp.where(mask0, src0 * CN + (dst0 - QN), 0)
    val = jnp.where(mask0, a0, 0.0)
    EEp_a = 51200
    fi = jnp.pad(fi, (0, EEp_a - E0))
    val = jnp.pad(val, (0, EEp_a - E0))
    zrow = jnp.zeros((_A0R,), jnp.float32)
    attn0_flat = _make_sc_attn0(E0, EEp_a)(fi, val, zrow)
    attn0 = attn0_flat[:QN * CN].reshape(QN, CN)

    X = W[QN:]
    s1 = edge_index[0]
    d1 = edge_index[1]
    EE1, EEp1 = s1.shape[0] + CN, 24576
    s1e, d1e = _pad_edges(s1, d1, CN, EEp1)

    h1 = X @ lin1_w
    h1h = h1.reshape(CN, HEADS, EMB)
    as1 = (h1h * att_src1).sum(-1).T
    ad1 = (h1h * att_dst1).sum(-1).T
    out1, _ = _sc_gat_layer(h1, as1, ad1, s1e, d1e,
                            NN=CN, NT=1008, H=HEADS, EE=EE1, EEp=EEp1,
                            CH=8192)
    X = jax.nn.relu(out1 + b1)

    h2 = X @ lin2_w
    h2h = h2.reshape(CN, HEADS, EMB)
    as2 = (h2h * att_src2).sum(-1).T
    ad2 = (h2h * att_dst2).sum(-1).T
    out2, _ = _sc_gat_layer(h2, as2, ad2, s1e, d1e,
                            NN=CN, NT=1008, H=HEADS, EE=EE1, EEp=EEp1,
                            CH=8192)
    X2 = out2.reshape(CN, HEADS, EMB).mean(axis=1) + b2

    routes9 = routes[..., :RSTEPS].reshape(-1)
    rt = X2[routes9].reshape(B * L, RSTEPS, EMB)
    idx = jnp.maximum(routes_len.reshape(-1) - 1, 0).astype(jnp.int32)
    wcat_r = jnp.concatenate([Wih_r.T, Whh_r.T], axis=0)
    br = (bih_r + bhh_r)[None, :]
    xt = _route_lstm(rt, idx, wcat_r, br, blk=800)
    xt = xt.reshape(B, L, HID)

    qt = W[questions]
    r = result_emb[answers]
    xcat = jnp.concatenate([qt, xt, r], axis=-1)
    xcat_lm = jnp.swapaxes(xcat, 0, 1)
    ind = attn0[questions[:, 1:]]
    ind_lm = jnp.swapaxes(ind, 0, 1)

    state_lm, pred_lm = _kt_chain(
        xcat_lm, fc0_w, fc0_b[None, :], Wih_k.T, Whh_k.T,
        (bih_k + bhh_k)[None, :], fc1_w, fc1_b[None, :], ind_lm)

    state = jnp.swapaxes(state_lm, 0, 1)
    pred = jnp.swapaxes(pred_lm, 0, 1)
    return (attn0, state, pred)


# speedup vs baseline: 2.0342x; 1.9541x over previous
import functools

import jax
import jax.numpy as jnp
from jax import lax
from jax.experimental import pallas as pl
from jax.experimental.pallas import tpu as pltpu, tpu_sc as plsc

QN = 4000
CN = 1000
N = QN + CN
EMB = 128
HID = 128
HEADS = 4
RSTEPS = 9

NTILES = 32
_MESH = plsc.VectorSubcoreMesh(core_axis_name="c", subcore_axis_name="s")
_SC_PARAMS = pltpu.CompilerParams(needs_layout_passes=False)


def _make_sc_gat(NN, NT, H, D, EE, EEp, CH, want_alpha):
    nd = D // NTILES
    n_chunks = EEp // CH
    grp = CH // 16
    tiles_per_head = NTILES // H
    na = EEp if want_alpha else CH

    @functools.partial(
        pl.kernel, mesh=_MESH, compiler_params=_SC_PARAMS,
        out_type=(jax.ShapeDtypeStruct((D * NT,), jnp.float32),
                  jax.ShapeDtypeStruct((NTILES * NT,), jnp.float32),
                  jax.ShapeDtypeStruct((NTILES * na,), jnp.float32)),
        scratch_types=[
            pltpu.VMEM((CH,), jnp.int32),
            pltpu.VMEM((CH,), jnp.int32),
            pltpu.VMEM((CH,), jnp.float32),
            pltpu.VMEM((NT,), jnp.float32),
            pltpu.VMEM((NT,), jnp.float32),
            pltpu.VMEM((NT,), jnp.float32),
            pltpu.VMEM((nd * NT,), jnp.float32),
            pltpu.VMEM((nd * NT,), jnp.float32),
        ],
    )
    def gat_edges(asrc_hbm, adst_hbm, h_hbm, src_hbm, dst_hbm,
                  out_hbm, den_hbm, al_hbm,
                  src_c, dst_c, al_c, asrc_v, adst_v, den_v, h_v, out_v):
        wid = lax.axis_index("s") * 2 + lax.axis_index("c")
        head = wid // tiles_per_head
        pltpu.sync_copy(asrc_hbm.at[pl.ds(head * NT, NT)], asrc_v)
        pltpu.sync_copy(adst_hbm.at[pl.ds(head * NT, NT)], adst_v)
        pltpu.sync_copy(h_hbm.at[pl.ds(wid * (nd * NT), nd * NT)], h_v)

        zero16 = jnp.zeros((16,), jnp.float32)

        def zloop(i, carry):
            den_v[pl.ds(i * 16, 16)] = zero16
            return carry

        lax.fori_loop(0, NT // 16, zloop, 0)

        def zloop2(i, carry):
            out_v[pl.ds(i * 16, 16)] = zero16
            return carry

        lax.fori_loop(0, nd * NT // 16, zloop2, 0)

        lane = lax.iota(jnp.int32, 16)

        def chunk_a(ci, carry):
            pltpu.sync_copy(src_hbm.at[pl.ds(ci * CH, CH)], src_c)
            pltpu.sync_copy(dst_hbm.at[pl.ds(ci * CH, CH)], dst_c)

            @plsc.parallel_loop(0, grp, unroll=4)
            def grp_a(g):
                s16 = src_c[pl.ds(g * 16, 16)]
                d16 = dst_c[pl.ds(g * 16, 16)]
                a = (plsc.load_gather(asrc_v, [s16])
                     + plsc.load_gather(adst_v, [d16]))
                a = jnp.where(a > 0, a, a * 0.2)
                ex = jnp.exp(a)
                mask = (ci * CH + g * 16 + lane) < EE
                plsc.addupdate_scatter(den_v, [d16], ex, mask=mask)

            return carry

        lax.fori_loop(0, n_chunks, chunk_a, 0)

        def chunk_b(ci, carry):
            pltpu.sync_copy(src_hbm.at[pl.ds(ci * CH, CH)], src_c)
            pltpu.sync_copy(dst_hbm.at[pl.ds(ci * CH, CH)], dst_c)

            @plsc.parallel_loop(0, grp, unroll=2)
            def grp_b(g):
                s16 = src_c[pl.ds(g * 16, 16)]
                d16 = dst_c[pl.ds(g * 16, 16)]
                a = (plsc.load_gather(asrc_v, [s16])
                     + plsc.load_gather(adst_v, [d16]))
                a = jnp.where(a > 0, a, a * 0.2)
                ex = jnp.exp(a)
                al = ex / plsc.load_gather(den_v, [d16])
                if want_alpha:
                    al_c[pl.ds(g * 16, 16)] = al
                mask = (ci * CH + g * 16 + lane) < EE
                for d in range(nd):
                    hv = plsc.load_gather(h_v, [s16 + d * NT])
                    plsc.addupdate_scatter(out_v, [d16 + d * NT],
                                           hv * al, mask=mask)

            if want_alpha:
                pltpu.sync_copy(
                    al_c, al_hbm.at[pl.ds(wid * EEp + ci * CH, CH)])
            return carry

        lax.fori_loop(0, n_chunks, chunk_b, 0)

        pltpu.sync_copy(out_v, out_hbm.at[pl.ds(wid * (nd * NT), nd * NT)])
        pltpu.sync_copy(den_v, den_hbm.at[pl.ds(wid * NT, NT)])

    return gat_edges


def _pad_nodes(x, NT):
    H, NN = x.shape
    return jnp.pad(x, ((0, 0), (0, NT - NN))).reshape(-1)


def _sc_gat_layer(h_nodes, a_src, a_dst, src_e, dst_e, NN, NT, H, EE, EEp,
                  CH, want_alpha=False):
    D = h_nodes.shape[1]
    h_flat = _pad_nodes(h_nodes.T, NT)
    asrc_flat = _pad_nodes(a_src, NT)
    adst_flat = _pad_nodes(a_dst, NT)
    fn = _make_sc_gat(NN, NT, H, D, EE, EEp, CH, want_alpha)
    out_flat, _, al_flat = fn(asrc_flat, adst_flat, h_flat, src_e, dst_e)
    out = out_flat.reshape(D, NT)[:, :NN].T
    alpha = al_flat[:EEp] if want_alpha else None
    return out, alpha


_A0R = 125008
_A0CH = 2048


def _make_sc_attn0(EE, EEp):
    n_chunks = EEp // _A0CH
    grp = _A0CH // 16

    @functools.partial(
        pl.kernel, mesh=_MESH, compiler_params=_SC_PARAMS,
        out_type=jax.ShapeDtypeStruct((NTILES * _A0R,), jnp.float32),
        scratch_types=[
            pltpu.VMEM((_A0CH,), jnp.int32),
            pltpu.VMEM((_A0CH,), jnp.float32),
            pltpu.VMEM((_A0R,), jnp.float32),
        ],
    )
    def attn0_scatter(fi_hbm, val_hbm, z_hbm, out_hbm, fi_c, val_c, tab_v):
        wid = lax.axis_index("s") * 2 + lax.axis_index("c")
        lo = wid * _A0R
        pltpu.sync_copy(z_hbm, tab_v)
        lane = lax.iota(jnp.int32, 16)

        def chunk(ci, carry):
            pltpu.sync_copy(fi_hbm.at[pl.ds(ci * _A0CH, _A0CH)], fi_c)
            pltpu.sync_copy(val_hbm.at[pl.ds(ci * _A0CH, _A0CH)], val_c)

            @plsc.parallel_loop(0, grp, unroll=4)
            def grp_f(g):
                f16 = fi_c[pl.ds(g * 16, 16)]
                v16 = val_c[pl.ds(g * 16, 16)]
                mask = ((f16 >= lo) & (f16 < lo + _A0R)
                        & ((ci * _A0CH + g * 16 + lane) < EE))
                loc = jnp.where(mask, f16 - lo, 0)
                plsc.addupdate_scatter(tab_v, [loc], v16, mask=mask)

            return carry

        lax.fori_loop(0, n_chunks, chunk, 0)
        pltpu.sync_copy(tab_v, out_hbm.at[pl.ds(lo, _A0R)])

    return attn0_scatter


def _route_lstm_body(rt_ref, idx_ref, wcat_ref, b_ref, out_ref):
    blk = rt_ref.shape[0]
    x = rt_ref[...]
    wcat = wcat_ref[...]
    b = b_ref[...]
    idx = idx_ref[...]
    h = jnp.zeros((blk, HID), jnp.float32)
    c = jnp.zeros((blk, HID), jnp.float32)
    out = jnp.zeros((blk, HID), jnp.float32)
    for t in range(RSTEPS):
        xt = x[:, t, :]
        g = jnp.concatenate([xt, h], axis=1) @ wcat + b
        i = jax.nn.sigmoid(g[:, :HID])
        f = jax.nn.sigmoid(g[:, HID:2 * HID])
        gg = jnp.tanh(g[:, 2 * HID:3 * HID])
        o = jax.nn.sigmoid(g[:, 3 * HID:])
        c = f * c + i * gg
        h = o * jnp.tanh(c)
        out = jnp.where(idx == t, h, out)
    out_ref[...] = out


def _route_lstm(rt, idx, wcat, b, blk):
    n = rt.shape[0]
    grid = n // blk
    idx2 = idx.reshape(n, 1)
    return pl.pallas_call(
        _route_lstm_body,
        grid=(grid,),
        in_specs=[
            pl.BlockSpec((blk, RSTEPS, EMB), lambda i: (i, 0, 0)),
            pl.BlockSpec((blk, 1), lambda i: (i, 0)),
            pl.BlockSpec((EMB + HID, 4 * HID), lambda i: (0, 0)),
            pl.BlockSpec((1, 4 * HID), lambda i: (0, 0)),
        ],
        out_specs=pl.BlockSpec((blk, HID), lambda i: (i, 0)),
        out_shape=jax.ShapeDtypeStruct((n, HID), jnp.float32),
    )(rt, idx2, wcat, b)


def _kt_body(xcat_ref, fc0w_ref, fc0b_ref, wih_ref, whh_ref, bk_ref,
             fc1w_ref, fc1b_ref, ind_ref, state_ref, pred_ref,
             gi_ref, hs_ref):
    L, B, F = xcat_ref.shape
    x0 = jax.nn.relu(
        xcat_ref[...].reshape(L * B, F) @ fc0w_ref[...] + fc0b_ref[...])
    gi_ref[...] = (x0 @ wih_ref[...] + bk_ref[...]).reshape(L, B, 4 * HID)

    whh = whh_ref[...]

    def step(t, hc):
        h, c = hc
        g = gi_ref[t] + h @ whh
        i = jax.nn.sigmoid(g[:, :HID])
        f = jax.nn.sigmoid(g[:, HID:2 * HID])
        gg = jnp.tanh(g[:, 2 * HID:3 * HID])
        o = jax.nn.sigmoid(g[:, 3 * HID:])
        c = f * c + i * gg
        h = o * jnp.tanh(c)
        hs_ref[t] = h
        return (h, c)

    h0 = jnp.zeros((B, HID), jnp.float32)
    c0 = jnp.zeros((B, HID), jnp.float32)
    jax.lax.fori_loop(0, L, step, (h0, c0))

    state = jax.nn.sigmoid(
        hs_ref[...].reshape(L * B, HID) @ fc1w_ref[...] + fc1b_ref[...])
    state = state.reshape(L, B, CN)
    state_ref[...] = state

    ind = ind_ref[...]
    whole = jnp.sum(ind, axis=-1)
    whole = jnp.where(whole > 0.0, whole, 1.0)
    pred_ref[...] = jnp.sum(state[:L - 1] * ind, axis=-1) / whole


def _kt_chain(xcat_lm, fc0w, fc0b, wih, whh, bk, fc1w, fc1b, ind_lm):
    L, B, F = xcat_lm.shape
    out_shape = (
        jax.ShapeDtypeStruct((L, B, CN), jnp.float32),
        jax.ShapeDtypeStruct((L - 1, B), jnp.float32),
    )
    return pl.pallas_call(
        _kt_body,
        out_shape=out_shape,
        scratch_shapes=[
            pltpu.VMEM((L, B, 4 * HID), jnp.float32),
            pltpu.VMEM((L, B, HID), jnp.float32),
        ],
    )(xcat_lm, fc0w, fc0b, wih, whh, bk, fc1w, fc1b, ind_lm)


def _pad_edges(src, dst, num_nodes, EEp):
    loops = jnp.arange(num_nodes, dtype=src.dtype)
    s = jnp.concatenate([src, loops])
    d = jnp.concatenate([dst, loops])
    pad = EEp - s.shape[0]
    return jnp.pad(s, (0, pad)), jnp.pad(d, (0, pad))


def kernel(students, questions, features, features_len, routes, routes_len,
           answers, whole_edge_index, whole_edge_attr, edge_index, edge_attr,
           lin0_w, att_src0, att_dst0, b0, lin1_w, att_src1, att_dst1, b1,
           lin2_w, att_src2, att_dst2, b2, result_emb, Wih_r, Whh_r, bih_r,
           bhh_r, fc0_w, fc0_b, Wih_k, Whh_k, bih_k, bhh_k, fc1_w, fc1_b):
    B, L = questions.shape

    h0 = lin0_w
    a_src0 = (h0 @ att_src0[0])[None, :]
    a_dst0 = (h0 @ att_dst0[0])[None, :]
    src0 = whole_edge_index[0]
    dst0 = whole_edge_index[1]
    E0 = src0.shape[0]
    EE0, EEp0 = E0 + N, 57344
    s0e, d0e = _pad_edges(src0, dst0, N, EEp0)
    out0, al0 = _sc_gat_layer(h0, a_src0, a_dst0, s0e, d0e,
                              NN=N, NT=5008, H=1, EE=EE0, EEp=EEp0, CH=8192,
                              want_alpha=True)
    W = out0 + b0

    a0 = al0[:E0]
    mask0 = (src0 < QN) & (dst0 >= QN)
    fi = jnp.where(mask0, src0 * CN + (dst0 - QN), 0)
    val = jnp.where(mask0, a0, 0.0)
    EEp_a = 51200
    fi = jnp.pad(fi, (0, EEp_a - E0))
    val = jnp.pad(val, (0, EEp_a - E0))
    zrow = jnp.zeros((_A0R,), jnp.float32)
    attn0_flat = _make_sc_attn0(E0, EEp_a)(fi, val, zrow)
    attn0 = attn0_flat[:QN * CN].reshape(QN, CN)

    X = W[QN:]
    s1 = edge_index[0]
    d1 = edge_index[1]
    EE1, EEp1 = s1.shape[0] + CN, 24576
    s1e, d1e = _pad_edges(s1, d1, CN, EEp1)

    h1 = X @ lin1_w
    h1h = h1.reshape(CN, HEADS, EMB)
    as1 = (h1h * att_src1).sum(-1).T
    ad1 = (h1h * att_dst1).sum(-1).T
    out1, _ = _sc_gat_layer(h1, as1, ad1, s1e, d1e,
                            NN=CN, NT=1008, H=HEADS, EE=EE1, EEp=EEp1,
                            CH=8192)
    X = jax.nn.relu(out1 + b1)

    h2 = X @ lin2_w
    h2h = h2.reshape(CN, HEADS, EMB)
    as2 = (h2h * att_src2).sum(-1).T
    ad2 = (h2h * att_dst2).sum(-1).T
    out2, _ = _sc_gat_layer(h2, as2, ad2, s1e, d1e,
                            NN=CN, NT=1008, H=HEADS, EE=EE1, EEp=EEp1,
                            CH=8192)
    X2 = out2.reshape(CN, HEADS, EMB).mean(axis=1) + b2

    routes9 = routes[..., :RSTEPS].reshape(-1)
    rt = X2[routes9].reshape(B * L, RSTEPS, EMB)
    idx = jnp.maximum(routes_len.reshape(-1) - 1, 0).astype(jnp.int32)
    wcat_r = jnp.concatenate([Wih_r.T, Whh_r.T], axis=0)
    br = (bih_r + bhh_r)[None, :]
    xt = _route_lstm(rt, idx, wcat_r, br, blk=800)
    xt = xt.reshape(B, L, HID)

    qt = W[questions]
    r = result_emb[answers]
    xcat = jnp.concatenate([qt, xt, r], axis=-1)
    xcat_lm = jnp.swapaxes(xcat, 0, 1)
    ind = attn0[questions[:, 1:]]
    ind_lm = jnp.swapaxes(ind, 0, 1)

    state_lm, pred_lm = _kt_chain(
        xcat_lm, fc0_w, fc0_b[None, :], Wih_k.T, Whh_k.T,
        (bih_k + bhh_k)[None, :], fc1_w, fc1_b[None, :], ind_lm)

    state = jnp.swapaxes(state_lm, 0, 1)
    pred = jnp.swapaxes(pred_lm, 0, 1)
    return (attn0, state, pred)


# speedup vs baseline: 2.0525x; 1.0090x over previous
import functools

import jax
import jax.numpy as jnp
from jax import lax
from jax.experimental import pallas as pl
from jax.experimental.pallas import tpu as pltpu, tpu_sc as plsc

QN = 4000
CN = 1000
N = QN + CN
EMB = 128
HID = 128
HEADS = 4
RSTEPS = 9

NTILES = 32
_MESH = plsc.VectorSubcoreMesh(core_axis_name="c", subcore_axis_name="s")
_SC_PARAMS = pltpu.CompilerParams(needs_layout_passes=False)


def _make_sc_gat(NN, NT, H, D, EE, EEp, CH, want_alpha):
    nd = D // NTILES
    n_chunks = EEp // CH
    grp = CH // 16
    tiles_per_head = NTILES // H
    na = EEp if want_alpha else CH

    @functools.partial(
        pl.kernel, mesh=_MESH, compiler_params=_SC_PARAMS,
        out_type=(jax.ShapeDtypeStruct((D * NT,), jnp.float32),
                  jax.ShapeDtypeStruct((NTILES * NT,), jnp.float32),
                  jax.ShapeDtypeStruct((NTILES * na,), jnp.float32)),
        scratch_types=[
            pltpu.VMEM((CH,), jnp.int32),
            pltpu.VMEM((CH,), jnp.int32),
            pltpu.VMEM((CH,), jnp.float32),
            pltpu.VMEM((NT,), jnp.float32),
            pltpu.VMEM((NT,), jnp.float32),
            pltpu.VMEM((NT,), jnp.float32),
            pltpu.VMEM((nd * NT,), jnp.float32),
            pltpu.VMEM((nd * NT,), jnp.float32),
        ],
    )
    def gat_edges(asrc_hbm, adst_hbm, h_hbm, src_hbm, dst_hbm,
                  out_hbm, den_hbm, al_hbm,
                  src_c, dst_c, al_c, asrc_v, adst_v, den_v, h_v, out_v):
        wid = lax.axis_index("s") * 2 + lax.axis_index("c")
        head = wid // tiles_per_head
        pltpu.sync_copy(asrc_hbm.at[pl.ds(head * NT, NT)], asrc_v)
        pltpu.sync_copy(adst_hbm.at[pl.ds(head * NT, NT)], adst_v)
        pltpu.sync_copy(h_hbm.at[pl.ds(wid * (nd * NT), nd * NT)], h_v)

        zero16 = jnp.zeros((16,), jnp.float32)

        def zloop(i, carry):
            den_v[pl.ds(i * 16, 16)] = zero16
            return carry

        lax.fori_loop(0, NT // 16, zloop, 0)

        def zloop2(i, carry):
            out_v[pl.ds(i * 16, 16)] = zero16
            return carry

        lax.fori_loop(0, nd * NT // 16, zloop2, 0)

        lane = lax.iota(jnp.int32, 16)

        def chunk_a(ci, carry):
            pltpu.sync_copy(src_hbm.at[pl.ds(ci * CH, CH)], src_c)
            pltpu.sync_copy(dst_hbm.at[pl.ds(ci * CH, CH)], dst_c)

            @plsc.parallel_loop(0, grp, unroll=4)
            def grp_a(g):
                s16 = src_c[pl.ds(g * 16, 16)]
                d16 = dst_c[pl.ds(g * 16, 16)]
                a = (plsc.load_gather(asrc_v, [s16])
                     + plsc.load_gather(adst_v, [d16]))
                a = jnp.where(a > 0, a, a * 0.2)
                ex = jnp.exp(a)
                mask = (ci * CH + g * 16 + lane) < EE
                plsc.addupdate_scatter(den_v, [d16], ex, mask=mask)

            return carry

        lax.fori_loop(0, n_chunks, chunk_a, 0)

        def chunk_b(ci, carry):
            pltpu.sync_copy(src_hbm.at[pl.ds(ci * CH, CH)], src_c)
            pltpu.sync_copy(dst_hbm.at[pl.ds(ci * CH, CH)], dst_c)

            @plsc.parallel_loop(0, grp, unroll=2)
            def grp_b(g):
                s16 = src_c[pl.ds(g * 16, 16)]
                d16 = dst_c[pl.ds(g * 16, 16)]
                a = (plsc.load_gather(asrc_v, [s16])
                     + plsc.load_gather(adst_v, [d16]))
                a = jnp.where(a > 0, a, a * 0.2)
                ex = jnp.exp(a)
                al = ex / plsc.load_gather(den_v, [d16])
                if want_alpha:
                    al_c[pl.ds(g * 16, 16)] = al
                mask = (ci * CH + g * 16 + lane) < EE
                for d in range(nd):
                    hv = plsc.load_gather(h_v, [s16 + d * NT])
                    plsc.addupdate_scatter(out_v, [d16 + d * NT],
                                           hv * al, mask=mask)

            if want_alpha:
                pltpu.sync_copy(
                    al_c, al_hbm.at[pl.ds(wid * EEp + ci * CH, CH)])
            return carry

        lax.fori_loop(0, n_chunks, chunk_b, 0)

        pltpu.sync_copy(out_v, out_hbm.at[pl.ds(wid * (nd * NT), nd * NT)])
        pltpu.sync_copy(den_v, den_hbm.at[pl.ds(wid * NT, NT)])

    return gat_edges


def _pad_nodes(x, NT):
    H, NN = x.shape
    return jnp.pad(x, ((0, 0), (0, NT - NN))).reshape(-1)


def _sc_gat_layer(h_nodes, a_src, a_dst, src_e, dst_e, NN, NT, H, EE, EEp,
                  CH, want_alpha=False):
    D = h_nodes.shape[1]
    h_flat = _pad_nodes(h_nodes.T, NT)
    asrc_flat = _pad_nodes(a_src, NT)
    adst_flat = _pad_nodes(a_dst, NT)
    fn = _make_sc_gat(NN, NT, H, D, EE, EEp, CH, want_alpha)
    out_flat, _, al_flat = fn(asrc_flat, adst_flat, h_flat, src_e, dst_e)
    out = out_flat.reshape(D, NT)[:, :NN].T
    alpha = al_flat[:EEp] if want_alpha else None
    return out, alpha


_A0R = 125008
_A0CH = 2048


def _make_sc_attn0(EE, EEp):
    n_chunks = EEp // _A0CH
    grp = _A0CH // 16

    @functools.partial(
        pl.kernel, mesh=_MESH, compiler_params=_SC_PARAMS,
        out_type=jax.ShapeDtypeStruct((NTILES * _A0R,), jnp.float32),
        scratch_types=[
            pltpu.VMEM((_A0CH,), jnp.int32),
            pltpu.VMEM((_A0CH,), jnp.float32),
            pltpu.VMEM((_A0R,), jnp.float32),
        ],
    )
    def attn0_scatter(fi_hbm, val_hbm, out_hbm, fi_c, val_c, tab_v):
        wid = lax.axis_index("s") * 2 + lax.axis_index("c")
        lo = wid * _A0R
        zero16 = jnp.zeros((16,), jnp.float32)

        @plsc.parallel_loop(0, _A0R // 16, unroll=8)
        def ztab(i):
            tab_v[pl.ds(i * 16, 16)] = zero16

        lane = lax.iota(jnp.int32, 16)

        def chunk(ci, carry):
            pltpu.sync_copy(fi_hbm.at[pl.ds(ci * _A0CH, _A0CH)], fi_c)
            pltpu.sync_copy(val_hbm.at[pl.ds(ci * _A0CH, _A0CH)], val_c)

            @plsc.parallel_loop(0, grp, unroll=4)
            def grp_f(g):
                f16 = fi_c[pl.ds(g * 16, 16)]
                v16 = val_c[pl.ds(g * 16, 16)]
                mask = ((f16 >= lo) & (f16 < lo + _A0R)
                        & ((ci * _A0CH + g * 16 + lane) < EE))
                loc = jnp.where(mask, f16 - lo, 0)
                plsc.addupdate_scatter(tab_v, [loc], v16, mask=mask)

            return carry

        lax.fori_loop(0, n_chunks, chunk, 0)
        pltpu.sync_copy(tab_v, out_hbm.at[pl.ds(lo, _A0R)])

    return attn0_scatter


def _route_lstm_body(rt_ref, idx_ref, wcat_ref, b_ref, out_ref):
    blk = rt_ref.shape[0]
    x = rt_ref[...]
    wcat = wcat_ref[...].astype(jnp.bfloat16)
    b = b_ref[...]
    idx = idx_ref[...]
    h = jnp.zeros((blk, HID), jnp.float32)
    c = jnp.zeros((blk, HID), jnp.float32)
    out = jnp.zeros((blk, HID), jnp.float32)
    for t in range(RSTEPS):
        xt = x[:, t, :]
        xh = jnp.concatenate([xt, h], axis=1).astype(jnp.bfloat16)
        g = jnp.dot(xh, wcat, preferred_element_type=jnp.float32) + b
        i = jax.nn.sigmoid(g[:, :HID])
        f = jax.nn.sigmoid(g[:, HID:2 * HID])
        gg = jnp.tanh(g[:, 2 * HID:3 * HID])
        o = jax.nn.sigmoid(g[:, 3 * HID:])
        c = f * c + i * gg
        h = o * jnp.tanh(c)
        out = jnp.where(idx == t, h, out)
    out_ref[...] = out


def _route_lstm(rt, idx, wcat, b, blk):
    n = rt.shape[0]
    grid = n // blk
    idx2 = idx.reshape(n, 1)
    return pl.pallas_call(
        _route_lstm_body,
        grid=(grid,),
        in_specs=[
            pl.BlockSpec((blk, RSTEPS, EMB), lambda i: (i, 0, 0)),
            pl.BlockSpec((blk, 1), lambda i: (i, 0)),
            pl.BlockSpec((EMB + HID, 4 * HID), lambda i: (0, 0)),
            pl.BlockSpec((1, 4 * HID), lambda i: (0, 0)),
        ],
        out_specs=pl.BlockSpec((blk, HID), lambda i: (i, 0)),
        out_shape=jax.ShapeDtypeStruct((n, HID), jnp.float32),
    )(rt, idx2, wcat, b)


def _kt_body(xcat_ref, fc0w_ref, fc0b_ref, wih_ref, whh_ref, bk_ref,
             fc1w_ref, fc1b_ref, ind_ref, state_ref, pred_ref,
             gi_ref, hs_ref):
    L, B, F = xcat_ref.shape
    bf = jnp.bfloat16
    x0 = jax.nn.relu(
        jnp.dot(xcat_ref[...].reshape(L * B, F).astype(bf),
                fc0w_ref[...].astype(bf),
                preferred_element_type=jnp.float32) + fc0b_ref[...])
    gi_ref[...] = (
        jnp.dot(x0.astype(bf), wih_ref[...].astype(bf),
                preferred_element_type=jnp.float32)
        + bk_ref[...]).reshape(L, B, 4 * HID)

    whh = whh_ref[...]

    def step(t, hc):
        h, c = hc
        g = gi_ref[t] + h @ whh
        i = jax.nn.sigmoid(g[:, :HID])
        f = jax.nn.sigmoid(g[:, HID:2 * HID])
        gg = jnp.tanh(g[:, 2 * HID:3 * HID])
        o = jax.nn.sigmoid(g[:, 3 * HID:])
        c = f * c + i * gg
        h = o * jnp.tanh(c)
        hs_ref[t] = h
        return (h, c)

    h0 = jnp.zeros((B, HID), jnp.float32)
    c0 = jnp.zeros((B, HID), jnp.float32)
    jax.lax.fori_loop(0, L, step, (h0, c0))

    state = jax.nn.sigmoid(
        jnp.dot(hs_ref[...].reshape(L * B, HID).astype(bf),
                fc1w_ref[...].astype(bf),
                preferred_element_type=jnp.float32) + fc1b_ref[...])
    state = state.reshape(L, B, CN)
    state_ref[...] = state

    ind = ind_ref[...]
    whole = jnp.sum(ind, axis=-1)
    whole = jnp.where(whole > 0.0, whole, 1.0)
    pred_ref[...] = jnp.sum(state[:L - 1] * ind, axis=-1) / whole


def _kt_chain(xcat_lm, fc0w, fc0b, wih, whh, bk, fc1w, fc1b, ind_lm):
    L, B, F = xcat_lm.shape
    out_shape = (
        jax.ShapeDtypeStruct((L, B, CN), jnp.float32),
        jax.ShapeDtypeStruct((L - 1, B), jnp.float32),
    )
    return pl.pallas_call(
        _kt_body,
        out_shape=out_shape,
        scratch_shapes=[
            pltpu.VMEM((L, B, 4 * HID), jnp.float32),
            pltpu.VMEM((L, B, HID), jnp.float32),
        ],
    )(xcat_lm, fc0w, fc0b, wih, whh, bk, fc1w, fc1b, ind_lm)


def _pad_edges(src, dst, num_nodes, EEp):
    loops = jnp.arange(num_nodes, dtype=src.dtype)
    s = jnp.concatenate([src, loops])
    d = jnp.concatenate([dst, loops])
    pad = EEp - s.shape[0]
    return jnp.pad(s, (0, pad)), jnp.pad(d, (0, pad))


def kernel(students, questions, features, features_len, routes, routes_len,
           answers, whole_edge_index, whole_edge_attr, edge_index, edge_attr,
           lin0_w, att_src0, att_dst0, b0, lin1_w, att_src1, att_dst1, b1,
           lin2_w, att_src2, att_dst2, b2, result_emb, Wih_r, Whh_r, bih_r,
           bhh_r, fc0_w, fc0_b, Wih_k, Whh_k, bih_k, bhh_k, fc1_w, fc1_b):
    B, L = questions.shape

    h0 = lin0_w
    a_src0 = (h0 @ att_src0[0])[None, :]
    a_dst0 = (h0 @ att_dst0[0])[None, :]
    src0 = whole_edge_index[0]
    dst0 = whole_edge_index[1]
    E0 = src0.shape[0]
    EE0, EEp0 = E0 + N, 57344
    s0e, d0e = _pad_edges(src0, dst0, N, EEp0)
    out0, al0 = _sc_gat_layer(h0, a_src0, a_dst0, s0e, d0e,
                              NN=N, NT=5008, H=1, EE=EE0, EEp=EEp0, CH=8192,
                              want_alpha=True)
    W = out0 + b0

    a0 = al0[:E0]
    mask0 = (src0 < QN) & (dst0 >= QN)
    fi = jnp.where(mask0, src0 * CN + (dst0 - QN), 0)
    val = jnp.where(mask0, a0, 0.0)
    EEp_a = 51200
    fi = jnp.pad(fi, (0, EEp_a - E0))
    val = jnp.pad(val, (0, EEp_a - E0))
    attn0_flat = _make_sc_attn0(E0, EEp_a)(fi, val)
    attn0 = attn0_flat[:QN * CN].reshape(QN, CN)

    X = W[QN:]
    s1 = edge_index[0]
    d1 = edge_index[1]
    EE1, EEp1 = s1.shape[0] + CN, 24576
    s1e, d1e = _pad_edges(s1, d1, CN, EEp1)

    h1 = X @ lin1_w
    h1h = h1.reshape(CN, HEADS, EMB)
    as1 = (h1h * att_src1).sum(-1).T
    ad1 = (h1h * att_dst1).sum(-1).T
    out1, _ = _sc_gat_layer(h1, as1, ad1, s1e, d1e,
                            NN=CN, NT=1008, H=HEADS, EE=EE1, EEp=EEp1,
                            CH=8192)
    X = jax.nn.relu(out1 + b1)

    h2 = X @ lin2_w
    h2h = h2.reshape(CN, HEADS, EMB)
    as2 = (h2h * att_src2).sum(-1).T
    ad2 = (h2h * att_dst2).sum(-1).T
    out2, _ = _sc_gat_layer(h2, as2, ad2, s1e, d1e,
                            NN=CN, NT=1008, H=HEADS, EE=EE1, EEp=EEp1,
                            CH=8192)
    X2 = out2.reshape(CN, HEADS, EMB).mean(axis=1) + b2

    routes9 = routes[..., :RSTEPS].reshape(-1)
    rt = X2[routes9].reshape(B * L, RSTEPS, EMB)
    idx = jnp.maximum(routes_len.reshape(-1) - 1, 0).astype(jnp.int32)
    wcat_r = jnp.concatenate([Wih_r.T, Whh_r.T], axis=0)
    br = (bih_r + bhh_r)[None, :]
    xt = _route_lstm(rt, idx, wcat_r, br, blk=800)
    xt = xt.reshape(B, L, HID)

    qt = W[questions]
    r = result_emb[answers]
    xcat = jnp.concatenate([qt, xt, r], axis=-1)
    xcat_lm = jnp.swapaxes(xcat, 0, 1)
    ind = attn0[questions[:, 1:]]
    ind_lm = jnp.swapaxes(ind, 0, 1)

    state_lm, pred_lm = _kt_chain(
        xcat_lm, fc0_w, fc0_b[None, :], Wih_k.T, Whh_k.T,
        (bih_k + bhh_k)[None, :], fc1_w, fc1_b[None, :], ind_lm)

    state = jnp.swapaxes(state_lm, 0, 1)
    pred = jnp.swapaxes(pred_lm, 0, 1)
    return (attn0, state, pred)


# speedup vs baseline: 2.0929x; 1.0197x over previous
import functools

import jax
import jax.numpy as jnp
from jax import lax
from jax.experimental import pallas as pl
from jax.experimental.pallas import tpu as pltpu, tpu_sc as plsc

QN = 4000
CN = 1000
N = QN + CN
EMB = 128
HID = 128
HEADS = 4
RSTEPS = 9

NTILES = 32
_MESH = plsc.VectorSubcoreMesh(core_axis_name="c", subcore_axis_name="s")
_SC_PARAMS = pltpu.CompilerParams(needs_layout_passes=False)


def _make_sc_gat(NN, NT, H, D, EE, EEp, CH, want_alpha):
    nd = D // NTILES
    n_chunks = EEp // CH
    grp = CH // 16
    tiles_per_head = NTILES // H
    na = EEp if want_alpha else CH

    @functools.partial(
        pl.kernel, mesh=_MESH, compiler_params=_SC_PARAMS,
        out_type=(jax.ShapeDtypeStruct((D * NT,), jnp.float32),
                  jax.ShapeDtypeStruct((NTILES * NT,), jnp.float32),
                  jax.ShapeDtypeStruct((NTILES * na,), jnp.float32)),
        scratch_types=[
            pltpu.VMEM((CH,), jnp.int32),
            pltpu.VMEM((CH,), jnp.int32),
            pltpu.VMEM((CH,), jnp.float32),
            pltpu.VMEM((NT,), jnp.float32),
            pltpu.VMEM((NT,), jnp.float32),
            pltpu.VMEM((NT,), jnp.float32),
            pltpu.VMEM((nd * NT,), jnp.float32),
            pltpu.VMEM((nd * NT,), jnp.float32),
        ],
    )
    def gat_edges(asrc_hbm, adst_hbm, h_hbm, src_hbm, dst_hbm,
                  out_hbm, den_hbm, al_hbm,
                  src_c, dst_c, al_c, asrc_v, adst_v, den_v, h_v, out_v):
        wid = lax.axis_index("s") * 2 + lax.axis_index("c")
        head = wid // tiles_per_head
        pltpu.sync_copy(asrc_hbm.at[pl.ds(head * NT, NT)], asrc_v)
        pltpu.sync_copy(adst_hbm.at[pl.ds(head * NT, NT)], adst_v)
        pltpu.sync_copy(h_hbm.at[pl.ds(wid * (nd * NT), nd * NT)], h_v)

        zero16 = jnp.zeros((16,), jnp.float32)

        def zloop(i, carry):
            den_v[pl.ds(i * 16, 16)] = zero16
            return carry

        lax.fori_loop(0, NT // 16, zloop, 0)

        def zloop2(i, carry):
            out_v[pl.ds(i * 16, 16)] = zero16
            return carry

        lax.fori_loop(0, nd * NT // 16, zloop2, 0)

        lane = lax.iota(jnp.int32, 16)

        def chunk_a(ci, carry):
            pltpu.sync_copy(src_hbm.at[pl.ds(ci * CH, CH)], src_c)
            pltpu.sync_copy(dst_hbm.at[pl.ds(ci * CH, CH)], dst_c)

            @plsc.parallel_loop(0, grp, unroll=4)
            def grp_a(g):
                s16 = src_c[pl.ds(g * 16, 16)]
                d16 = dst_c[pl.ds(g * 16, 16)]
                a = (plsc.load_gather(asrc_v, [s16])
                     + plsc.load_gather(adst_v, [d16]))
                a = jnp.where(a > 0, a, a * 0.2)
                ex = jnp.exp(a)
                mask = (ci * CH + g * 16 + lane) < EE
                plsc.addupdate_scatter(den_v, [d16], ex, mask=mask)

            return carry

        lax.fori_loop(0, n_chunks, chunk_a, 0)

        def chunk_b(ci, carry):
            pltpu.sync_copy(src_hbm.at[pl.ds(ci * CH, CH)], src_c)
            pltpu.sync_copy(dst_hbm.at[pl.ds(ci * CH, CH)], dst_c)

            @plsc.parallel_loop(0, grp, unroll=2)
            def grp_b(g):
                s16 = src_c[pl.ds(g * 16, 16)]
                d16 = dst_c[pl.ds(g * 16, 16)]
                a = (plsc.load_gather(asrc_v, [s16])
                     + plsc.load_gather(adst_v, [d16]))
                a = jnp.where(a > 0, a, a * 0.2)
                ex = jnp.exp(a)
                al = ex / plsc.load_gather(den_v, [d16])
                if want_alpha:
                    al_c[pl.ds(g * 16, 16)] = al
                mask = (ci * CH + g * 16 + lane) < EE
                for d in range(nd):
                    hv = plsc.load_gather(h_v, [s16 + d * NT])
                    plsc.addupdate_scatter(out_v, [d16 + d * NT],
                                           hv * al, mask=mask)

            if want_alpha:
                pltpu.sync_copy(
                    al_c, al_hbm.at[pl.ds(wid * EEp + ci * CH, CH)])
            return carry

        lax.fori_loop(0, n_chunks, chunk_b, 0)

        pltpu.sync_copy(out_v, out_hbm.at[pl.ds(wid * (nd * NT), nd * NT)])
        pltpu.sync_copy(den_v, den_hbm.at[pl.ds(wid * NT, NT)])

    return gat_edges


def _pad_nodes(x, NT):
    H, NN = x.shape
    return jnp.pad(x, ((0, 0), (0, NT - NN))).reshape(-1)


def _sc_gat_layer(h_nodes, a_src, a_dst, src_e, dst_e, NN, NT, H, EE, EEp,
                  CH, want_alpha=False):
    D = h_nodes.shape[1]
    h_flat = _pad_nodes(h_nodes.T, NT)
    asrc_flat = _pad_nodes(a_src, NT)
    adst_flat = _pad_nodes(a_dst, NT)
    fn = _make_sc_gat(NN, NT, H, D, EE, EEp, CH, want_alpha)
    out_flat, _, al_flat = fn(asrc_flat, adst_flat, h_flat, src_e, dst_e)
    out = out_flat.reshape(D, NT)[:, :NN].T
    alpha = al_flat[:EEp] if want_alpha else None
    return out, alpha


_A0R = 125008
_A0CH = 2048


def _make_sc_attn0(EE, EEp):
    n_chunks = EEp // _A0CH
    grp = _A0CH // 16

    @functools.partial(
        pl.kernel, mesh=_MESH, compiler_params=_SC_PARAMS,
        out_type=jax.ShapeDtypeStruct((NTILES * _A0R,), jnp.float32),
        scratch_types=[
            pltpu.VMEM((_A0CH,), jnp.int32),
            pltpu.VMEM((_A0CH,), jnp.float32),
            pltpu.VMEM((_A0R,), jnp.float32),
        ],
    )
    def attn0_scatter(fi_hbm, val_hbm, out_hbm, fi_c, val_c, tab_v):
        wid = lax.axis_index("s") * 2 + lax.axis_index("c")
        lo = wid * _A0R
        zero16 = jnp.zeros((16,), jnp.float32)

        @plsc.parallel_loop(0, _A0R // 16, unroll=8)
        def ztab(i):
            tab_v[pl.ds(i * 16, 16)] = zero16

        lane = lax.iota(jnp.int32, 16)

        def chunk(ci, carry):
            pltpu.sync_copy(fi_hbm.at[pl.ds(ci * _A0CH, _A0CH)], fi_c)
            pltpu.sync_copy(val_hbm.at[pl.ds(ci * _A0CH, _A0CH)], val_c)

            @plsc.parallel_loop(0, grp, unroll=4)
            def grp_f(g):
                f16 = fi_c[pl.ds(g * 16, 16)]
                v16 = val_c[pl.ds(g * 16, 16)]
                mask = ((f16 >= lo) & (f16 < lo + _A0R)
                        & ((ci * _A0CH + g * 16 + lane) < EE))
                loc = jnp.where(mask, f16 - lo, 0)
                plsc.addupdate_scatter(tab_v, [loc], v16, mask=mask)

            return carry

        lax.fori_loop(0, n_chunks, chunk, 0)
        pltpu.sync_copy(tab_v, out_hbm.at[pl.ds(lo, _A0R)])

    return attn0_scatter


def _route_lstm_body(rt_ref, idx_ref, wcat_ref, b_ref, out_ref):
    blk = rt_ref.shape[0]
    x = rt_ref[...]
    wcat = wcat_ref[...].astype(jnp.bfloat16)
    b = b_ref[...]
    idx = idx_ref[...]
    h = jnp.zeros((blk, HID), jnp.float32)
    c = jnp.zeros((blk, HID), jnp.float32)
    out = jnp.zeros((blk, HID), jnp.float32)
    for t in range(RSTEPS):
        xt = x[:, t, :]
        xh = jnp.concatenate([xt, h], axis=1).astype(jnp.bfloat16)
        g = jnp.dot(xh, wcat, preferred_element_type=jnp.float32) + b
        i = jax.nn.sigmoid(g[:, :HID])
        f = jax.nn.sigmoid(g[:, HID:2 * HID])
        gg = jnp.tanh(g[:, 2 * HID:3 * HID])
        o = jax.nn.sigmoid(g[:, 3 * HID:])
        c = f * c + i * gg
        h = o * jnp.tanh(c)
        out = jnp.where(idx == t, h, out)
    out_ref[...] = out


def _route_lstm(rt, idx, wcat, b, blk):
    n = rt.shape[0]
    grid = n // blk
    idx2 = idx.reshape(n, 1)
    return pl.pallas_call(
        _route_lstm_body,
        grid=(grid,),
        in_specs=[
            pl.BlockSpec((blk, RSTEPS, EMB), lambda i: (i, 0, 0)),
            pl.BlockSpec((blk, 1), lambda i: (i, 0)),
            pl.BlockSpec((EMB + HID, 4 * HID), lambda i: (0, 0)),
            pl.BlockSpec((1, 4 * HID), lambda i: (0, 0)),
        ],
        out_specs=pl.BlockSpec((blk, HID), lambda i: (i, 0)),
        out_shape=jax.ShapeDtypeStruct((n, HID), jnp.float32),
    )(rt, idx2, wcat, b)


def _kt_body(xcat_ref, fc0w_ref, fc0b_ref, wih_ref, whh_ref, bk_ref,
             fc1w_ref, fc1b_ref, ind_ref, state_ref, pred_ref,
             gi_ref, hs_ref):
    L, B, F = xcat_ref.shape
    bf = jnp.bfloat16
    x0 = jax.nn.relu(
        jnp.dot(xcat_ref[...].reshape(L * B, F).astype(bf),
                fc0w_ref[...].astype(bf),
                preferred_element_type=jnp.float32) + fc0b_ref[...])
    gi_ref[...] = (
        jnp.dot(x0.astype(bf), wih_ref[...].astype(bf),
                preferred_element_type=jnp.float32)
        + bk_ref[...]).reshape(L, B, 4 * HID)

    whh = whh_ref[...]

    def step(t, hc):
        h, c = hc
        g = gi_ref[t] + h @ whh
        i = jax.nn.sigmoid(g[:, :HID])
        f = jax.nn.sigmoid(g[:, HID:2 * HID])
        gg = jnp.tanh(g[:, 2 * HID:3 * HID])
        o = jax.nn.sigmoid(g[:, 3 * HID:])
        c = f * c + i * gg
        h = o * jnp.tanh(c)
        hs_ref[t] = h
        return (h, c)

    h0 = jnp.zeros((B, HID), jnp.float32)
    c0 = jnp.zeros((B, HID), jnp.float32)
    jax.lax.fori_loop(0, L, step, (h0, c0))

    state = jax.nn.sigmoid(
        jnp.dot(hs_ref[...].reshape(L * B, HID).astype(bf),
                fc1w_ref[...].astype(bf),
                preferred_element_type=jnp.float32) + fc1b_ref[...])
    state = state.reshape(L, B, CN)
    state_ref[...] = state

    ind = ind_ref[...]
    whole = jnp.sum(ind, axis=-1)
    whole = jnp.where(whole > 0.0, whole, 1.0)
    pred_ref[...] = jnp.sum(state[:L - 1] * ind, axis=-1) / whole


def _kt_chain(xcat_lm, fc0w, fc0b, wih, whh, bk, fc1w, fc1b, ind_lm):
    L, B, F = xcat_lm.shape
    out_shape = (
        jax.ShapeDtypeStruct((L, B, CN), jnp.float32),
        jax.ShapeDtypeStruct((L - 1, B), jnp.float32),
    )
    return pl.pallas_call(
        _kt_body,
        out_shape=out_shape,
        scratch_shapes=[
            pltpu.VMEM((L, B, 4 * HID), jnp.float32),
            pltpu.VMEM((L, B, HID), jnp.float32),
        ],
    )(xcat_lm, fc0w, fc0b, wih, whh, bk, fc1w, fc1b, ind_lm)


def _pad_edges(src, dst, num_nodes, EEp):
    loops = jnp.arange(num_nodes, dtype=src.dtype)
    s = jnp.concatenate([src, loops])
    d = jnp.concatenate([dst, loops])
    pad = EEp - s.shape[0]
    return jnp.pad(s, (0, pad)), jnp.pad(d, (0, pad))


def kernel(students, questions, features, features_len, routes, routes_len,
           answers, whole_edge_index, whole_edge_attr, edge_index, edge_attr,
           lin0_w, att_src0, att_dst0, b0, lin1_w, att_src1, att_dst1, b1,
           lin2_w, att_src2, att_dst2, b2, result_emb, Wih_r, Whh_r, bih_r,
           bhh_r, fc0_w, fc0_b, Wih_k, Whh_k, bih_k, bhh_k, fc1_w, fc1_b):
    B, L = questions.shape

    h0 = lin0_w
    a_src0 = (h0 @ att_src0[0])[None, :]
    a_dst0 = (h0 @ att_dst0[0])[None, :]
    src0 = whole_edge_index[0]
    dst0 = whole_edge_index[1]
    E0 = src0.shape[0]
    EE0, EEp0 = E0 + N, 57344
    s0e, d0e = _pad_edges(src0, dst0, N, EEp0)
    out0, al0 = _sc_gat_layer(h0, a_src0, a_dst0, s0e, d0e,
                              NN=N, NT=5008, H=1, EE=EE0, EEp=EEp0, CH=14336,
                              want_alpha=True)
    W = out0 + b0

    a0 = al0[:E0]
    mask0 = (src0 < QN) & (dst0 >= QN)
    fi = jnp.where(mask0, src0 * CN + (dst0 - QN), 0)
    val = jnp.where(mask0, a0, 0.0)
    EEp_a = 51200
    fi = jnp.pad(fi, (0, EEp_a - E0))
    val = jnp.pad(val, (0, EEp_a - E0))
    attn0_flat = _make_sc_attn0(E0, EEp_a)(fi, val)
    attn0 = attn0_flat[:QN * CN].reshape(QN, CN)

    X = W[QN:]
    s1 = edge_index[0]
    d1 = edge_index[1]
    EE1, EEp1 = s1.shape[0] + CN, 24576
    s1e, d1e = _pad_edges(s1, d1, CN, EEp1)

    h1 = X @ lin1_w
    h1h = h1.reshape(CN, HEADS, EMB)
    as1 = (h1h * att_src1).sum(-1).T
    ad1 = (h1h * att_dst1).sum(-1).T
    out1, _ = _sc_gat_layer(h1, as1, ad1, s1e, d1e,
                            NN=CN, NT=1008, H=HEADS, EE=EE1, EEp=EEp1,
                            CH=24576)
    X = jax.nn.relu(out1 + b1)

    h2 = X @ lin2_w
    h2h = h2.reshape(CN, HEADS, EMB)
    as2 = (h2h * att_src2).sum(-1).T
    ad2 = (h2h * att_dst2).sum(-1).T
    out2, _ = _sc_gat_layer(h2, as2, ad2, s1e, d1e,
                            NN=CN, NT=1008, H=HEADS, EE=EE1, EEp=EEp1,
                            CH=24576)
    X2 = out2.reshape(CN, HEADS, EMB).mean(axis=1) + b2

    routes9 = routes[..., :RSTEPS].reshape(-1)
    rt = X2[routes9].reshape(B * L, RSTEPS, EMB)
    idx = jnp.maximum(routes_len.reshape(-1) - 1, 0).astype(jnp.int32)
    wcat_r = jnp.concatenate([Wih_r.T, Whh_r.T], axis=0)
    br = (bih_r + bhh_r)[None, :]
    xt = _route_lstm(rt, idx, wcat_r, br, blk=800)
    xt = xt.reshape(B, L, HID)

    qt = W[questions]
    r = result_emb[answers]
    xcat = jnp.concatenate([qt, xt, r], axis=-1)
    xcat_lm = jnp.swapaxes(xcat, 0, 1)
    ind = attn0[questions[:, 1:]]
    ind_lm = jnp.swapaxes(ind, 0, 1)

    state_lm, pred_lm = _kt_chain(
        xcat_lm, fc0_w, fc0_b[None, :], Wih_k.T, Whh_k.T,
        (bih_k + bhh_k)[None, :], fc1_w, fc1_b[None, :], ind_lm)

    state = jnp.swapaxes(state_lm, 0, 1)
    pred = jnp.swapaxes(pred_lm, 0, 1)
    return (attn0, state, pred)


# speedup vs baseline: 2.1314x; 1.0184x over previous
import functools

import jax
import jax.numpy as jnp
from jax import lax
from jax.experimental import pallas as pl
from jax.experimental.pallas import tpu as pltpu, tpu_sc as plsc

QN = 4000
CN = 1000
N = QN + CN
EMB = 128
HID = 128
HEADS = 4
RSTEPS = 9

NTILES = 32
_MESH = plsc.VectorSubcoreMesh(core_axis_name="c", subcore_axis_name="s")
_SC_PARAMS = pltpu.CompilerParams(needs_layout_passes=False)


def _make_sc_gat(NN, NT, H, D, EE, EEp, CH, want_alpha):
    nd = D // NTILES
    n_chunks = EEp // CH
    grp = CH // 16
    tiles_per_head = NTILES // H
    na = EEp if want_alpha else CH

    @functools.partial(
        pl.kernel, mesh=_MESH, compiler_params=_SC_PARAMS,
        out_type=(jax.ShapeDtypeStruct((D * NT,), jnp.float32),
                  jax.ShapeDtypeStruct((NTILES * NT,), jnp.float32),
                  jax.ShapeDtypeStruct((NTILES * na,), jnp.float32)),
        scratch_types=[
            pltpu.VMEM((CH,), jnp.int32),
            pltpu.VMEM((CH,), jnp.int32),
            pltpu.VMEM((CH,), jnp.float32),
            pltpu.VMEM((NT,), jnp.float32),
            pltpu.VMEM((NT,), jnp.float32),
            pltpu.VMEM((NT,), jnp.float32),
            pltpu.VMEM((nd * NT,), jnp.float32),
            pltpu.VMEM((nd * NT,), jnp.float32),
        ],
    )
    def gat_edges(asrc_hbm, adst_hbm, h_hbm, src_hbm, dst_hbm,
                  out_hbm, den_hbm, al_hbm,
                  src_c, dst_c, al_c, asrc_v, adst_v, den_v, h_v, out_v):
        wid = lax.axis_index("s") * 2 + lax.axis_index("c")
        head = wid // tiles_per_head
        pltpu.sync_copy(asrc_hbm.at[pl.ds(head * NT, NT)], asrc_v)
        pltpu.sync_copy(adst_hbm.at[pl.ds(head * NT, NT)], adst_v)
        pltpu.sync_copy(h_hbm.at[pl.ds(wid * (nd * NT), nd * NT)], h_v)

        zero16 = jnp.zeros((16,), jnp.float32)

        def zloop(i, carry):
            den_v[pl.ds(i * 16, 16)] = zero16
            return carry

        lax.fori_loop(0, NT // 16, zloop, 0)

        def zloop2(i, carry):
            out_v[pl.ds(i * 16, 16)] = zero16
            return carry

        lax.fori_loop(0, nd * NT // 16, zloop2, 0)

        lane = lax.iota(jnp.int32, 16)

        def chunk_a(ci, carry):
            pltpu.sync_copy(src_hbm.at[pl.ds(ci * CH, CH)], src_c)
            pltpu.sync_copy(dst_hbm.at[pl.ds(ci * CH, CH)], dst_c)

            @plsc.parallel_loop(0, grp, unroll=4)
            def grp_a(g):
                s16 = src_c[pl.ds(g * 16, 16)]
                d16 = dst_c[pl.ds(g * 16, 16)]
                a = (plsc.load_gather(asrc_v, [s16])
                     + plsc.load_gather(adst_v, [d16]))
                a = jnp.where(a > 0, a, a * 0.2)
                ex = jnp.exp(a)
                mask = (ci * CH + g * 16 + lane) < EE
                plsc.addupdate_scatter(den_v, [d16], ex, mask=mask)

            return carry

        lax.fori_loop(0, n_chunks, chunk_a, 0)

        def chunk_b(ci, carry):
            pltpu.sync_copy(src_hbm.at[pl.ds(ci * CH, CH)], src_c)
            pltpu.sync_copy(dst_hbm.at[pl.ds(ci * CH, CH)], dst_c)

            @plsc.parallel_loop(0, grp, unroll=2)
            def grp_b(g):
                s16 = src_c[pl.ds(g * 16, 16)]
                d16 = dst_c[pl.ds(g * 16, 16)]
                a = (plsc.load_gather(asrc_v, [s16])
                     + plsc.load_gather(adst_v, [d16]))
                a = jnp.where(a > 0, a, a * 0.2)
                ex = jnp.exp(a)
                al = ex / plsc.load_gather(den_v, [d16])
                if want_alpha:
                    al_c[pl.ds(g * 16, 16)] = al
                mask = (ci * CH + g * 16 + lane) < EE
                for d in range(nd):
                    hv = plsc.load_gather(h_v, [s16 + d * NT])
                    plsc.addupdate_scatter(out_v, [d16 + d * NT],
                                           hv * al, mask=mask)

            if want_alpha:
                pltpu.sync_copy(
                    al_c, al_hbm.at[pl.ds(wid * EEp + ci * CH, CH)])
            return carry

        lax.fori_loop(0, n_chunks, chunk_b, 0)

        pltpu.sync_copy(out_v, out_hbm.at[pl.ds(wid * (nd * NT), nd * NT)])
        pltpu.sync_copy(den_v, den_hbm.at[pl.ds(wid * NT, NT)])

    return gat_edges


def _pad_nodes(x, NT):
    H, NN = x.shape
    return jnp.pad(x, ((0, 0), (0, NT - NN))).reshape(-1)


def _sc_gat_layer(h_nodes, a_src, a_dst, src_e, dst_e, NN, NT, H, EE, EEp,
                  CH, want_alpha=False):
    D = h_nodes.shape[1]
    h_flat = _pad_nodes(h_nodes.T, NT)
    asrc_flat = _pad_nodes(a_src, NT)
    adst_flat = _pad_nodes(a_dst, NT)
    fn = _make_sc_gat(NN, NT, H, D, EE, EEp, CH, want_alpha)
    out_flat, _, al_flat = fn(asrc_flat, adst_flat, h_flat, src_e, dst_e)
    out = out_flat.reshape(D, NT)[:, :NN].T
    alpha = al_flat[:EEp] if want_alpha else None
    return out, alpha


_A0R = 125008
_A0CH = 2048


def _make_sc_attn0(EE, EEp):
    n_chunks = EEp // _A0CH
    grp = _A0CH // 16

    @functools.partial(
        pl.kernel, mesh=_MESH, compiler_params=_SC_PARAMS,
        out_type=jax.ShapeDtypeStruct((NTILES * _A0R,), jnp.float32),
        scratch_types=[
            pltpu.VMEM((_A0CH,), jnp.int32),
            pltpu.VMEM((_A0CH,), jnp.float32),
            pltpu.VMEM((_A0R,), jnp.float32),
        ],
    )
    def attn0_scatter(fi_hbm, val_hbm, out_hbm, fi_c, val_c, tab_v):
        wid = lax.axis_index("s") * 2 + lax.axis_index("c")
        lo = wid * _A0R
        zero16 = jnp.zeros((16,), jnp.float32)

        @plsc.parallel_loop(0, _A0R // 16, unroll=8)
        def ztab(i):
            tab_v[pl.ds(i * 16, 16)] = zero16

        lane = lax.iota(jnp.int32, 16)

        def chunk(ci, carry):
            pltpu.sync_copy(fi_hbm.at[pl.ds(ci * _A0CH, _A0CH)], fi_c)
            pltpu.sync_copy(val_hbm.at[pl.ds(ci * _A0CH, _A0CH)], val_c)

            @plsc.parallel_loop(0, grp, unroll=4)
            def grp_f(g):
                f16 = fi_c[pl.ds(g * 16, 16)]
                v16 = val_c[pl.ds(g * 16, 16)]
                mask = ((f16 >= lo) & (f16 < lo + _A0R)
                        & ((ci * _A0CH + g * 16 + lane) < EE))
                loc = jnp.where(mask, f16 - lo, 0)
                plsc.addupdate_scatter(tab_v, [loc], v16, mask=mask)

            return carry

        lax.fori_loop(0, n_chunks, chunk, 0)
        pltpu.sync_copy(tab_v, out_hbm.at[pl.ds(lo, _A0R)])

    return attn0_scatter


def _route_lstm_body(rt_ref, idx_ref, wcat_ref, b_ref, out_ref):
    blk = rt_ref.shape[0]
    x = rt_ref[...]
    wcat = wcat_ref[...].astype(jnp.bfloat16)
    b = b_ref[...]
    idx = idx_ref[...]
    h = jnp.zeros((blk, HID), jnp.float32)
    c = jnp.zeros((blk, HID), jnp.float32)
    out = jnp.zeros((blk, HID), jnp.float32)
    for t in range(RSTEPS):
        xt = x[:, t, :]
        xh = jnp.concatenate([xt, h], axis=1).astype(jnp.bfloat16)
        g = jnp.dot(xh, wcat, preferred_element_type=jnp.float32) + b
        i = jax.nn.sigmoid(g[:, :HID])
        f = jax.nn.sigmoid(g[:, HID:2 * HID])
        gg = jnp.tanh(g[:, 2 * HID:3 * HID])
        o = jax.nn.sigmoid(g[:, 3 * HID:])
        c = f * c + i * gg
        h = o * jnp.tanh(c)
        out = jnp.where(idx == t, h, out)
    out_ref[...] = out


def _route_lstm(rt, idx, wcat, b, blk):
    n = rt.shape[0]
    grid = n // blk
    idx2 = idx.reshape(n, 1)
    return pl.pallas_call(
        _route_lstm_body,
        grid=(grid,),
        in_specs=[
            pl.BlockSpec((blk, RSTEPS, EMB), lambda i: (i, 0, 0)),
            pl.BlockSpec((blk, 1), lambda i: (i, 0)),
            pl.BlockSpec((EMB + HID, 4 * HID), lambda i: (0, 0)),
            pl.BlockSpec((1, 4 * HID), lambda i: (0, 0)),
        ],
        out_specs=pl.BlockSpec((blk, HID), lambda i: (i, 0)),
        out_shape=jax.ShapeDtypeStruct((n, HID), jnp.float32),
    )(rt, idx2, wcat, b)


def _kt_body(xcat_ref, fc0w_ref, fc0b_ref, wih_ref, whh_ref, bk_ref,
             fc1w_ref, fc1b_ref, ind_ref, state_ref, pred_ref,
             gi_ref, hs_ref):
    B, L, F = xcat_ref.shape
    bf = jnp.bfloat16
    x0 = jax.nn.relu(
        jnp.dot(xcat_ref[...].reshape(B * L, F).astype(bf),
                fc0w_ref[...].astype(bf),
                preferred_element_type=jnp.float32) + fc0b_ref[...])
    gi_ref[...] = (
        jnp.dot(x0.astype(bf), wih_ref[...].astype(bf),
                preferred_element_type=jnp.float32)
        + bk_ref[...]).reshape(B, L, 4 * HID)

    whh = whh_ref[...]

    def step(t, hc):
        h, c = hc
        g = gi_ref[:, t] + h @ whh
        i = jax.nn.sigmoid(g[:, :HID])
        f = jax.nn.sigmoid(g[:, HID:2 * HID])
        gg = jnp.tanh(g[:, 2 * HID:3 * HID])
        o = jax.nn.sigmoid(g[:, 3 * HID:])
        c = f * c + i * gg
        h = o * jnp.tanh(c)
        hs_ref[:, t] = h
        return (h, c)

    h0 = jnp.zeros((B, HID), jnp.float32)
    c0 = jnp.zeros((B, HID), jnp.float32)
    jax.lax.fori_loop(0, L, step, (h0, c0))

    state = jax.nn.sigmoid(
        jnp.dot(hs_ref[...].reshape(B * L, HID).astype(bf),
                fc1w_ref[...].astype(bf),
                preferred_element_type=jnp.float32) + fc1b_ref[...])
    state = state.reshape(B, L, CN)
    state_ref[...] = state

    ind = ind_ref[...]
    whole = jnp.sum(ind, axis=-1)
    whole = jnp.where(whole > 0.0, whole, 1.0)
    pred_ref[...] = jnp.sum(state[:, :L - 1] * ind, axis=-1) / whole


def _kt_chain(xcat, fc0w, fc0b, wih, whh, bk, fc1w, fc1b, ind):
    B, L, F = xcat.shape
    out_shape = (
        jax.ShapeDtypeStruct((B, L, CN), jnp.float32),
        jax.ShapeDtypeStruct((B, L - 1), jnp.float32),
    )
    return pl.pallas_call(
        _kt_body,
        out_shape=out_shape,
        scratch_shapes=[
            pltpu.VMEM((B, L, 4 * HID), jnp.float32),
            pltpu.VMEM((B, L, HID), jnp.float32),
        ],
    )(xcat, fc0w, fc0b, wih, whh, bk, fc1w, fc1b, ind)


def _pad_edges(src, dst, num_nodes, EEp):
    loops = jnp.arange(num_nodes, dtype=src.dtype)
    s = jnp.concatenate([src, loops])
    d = jnp.concatenate([dst, loops])
    pad = EEp - s.shape[0]
    return jnp.pad(s, (0, pad)), jnp.pad(d, (0, pad))


def kernel(students, questions, features, features_len, routes, routes_len,
           answers, whole_edge_index, whole_edge_attr, edge_index, edge_attr,
           lin0_w, att_src0, att_dst0, b0, lin1_w, att_src1, att_dst1, b1,
           lin2_w, att_src2, att_dst2, b2, result_emb, Wih_r, Whh_r, bih_r,
           bhh_r, fc0_w, fc0_b, Wih_k, Whh_k, bih_k, bhh_k, fc1_w, fc1_b):
    B, L = questions.shape

    h0 = lin0_w
    a_src0 = (h0 @ att_src0[0])[None, :]
    a_dst0 = (h0 @ att_dst0[0])[None, :]
    src0 = whole_edge_index[0]
    dst0 = whole_edge_index[1]
    E0 = src0.shape[0]
    EE0, EEp0 = E0 + N, 57344
    s0e, d0e = _pad_edges(src0, dst0, N, EEp0)
    out0, al0 = _sc_gat_layer(h0, a_src0, a_dst0, s0e, d0e,
                              NN=N, NT=5008, H=1, EE=EE0, EEp=EEp0, CH=14336,
                              want_alpha=True)
    W = out0 + b0

    a0 = al0[:E0]
    mask0 = (src0 < QN) & (dst0 >= QN)
    fi = jnp.where(mask0, src0 * CN + (dst0 - QN), 0)
    val = jnp.where(mask0, a0, 0.0)
    EEp_a = 51200
    fi = jnp.pad(fi, (0, EEp_a - E0))
    val = jnp.pad(val, (0, EEp_a - E0))
    attn0_flat = _make_sc_attn0(E0, EEp_a)(fi, val)
    attn0 = attn0_flat[:QN * CN].reshape(QN, CN)

    X = W[QN:]
    s1 = edge_index[0]
    d1 = edge_index[1]
    EE1, EEp1 = s1.shape[0] + CN, 24576
    s1e, d1e = _pad_edges(s1, d1, CN, EEp1)

    h1 = X @ lin1_w
    h1h = h1.reshape(CN, HEADS, EMB)
    as1 = (h1h * att_src1).sum(-1).T
    ad1 = (h1h * att_dst1).sum(-1).T
    out1, _ = _sc_gat_layer(h1, as1, ad1, s1e, d1e,
                            NN=CN, NT=1008, H=HEADS, EE=EE1, EEp=EEp1,
                            CH=24576)
    X = jax.nn.relu(out1 + b1)

    h2 = X @ lin2_w
    h2h = h2.reshape(CN, HEADS, EMB)
    as2 = (h2h * att_src2).sum(-1).T
    ad2 = (h2h * att_dst2).sum(-1).T
    out2, _ = _sc_gat_layer(h2, as2, ad2, s1e, d1e,
                            NN=CN, NT=1008, H=HEADS, EE=EE1, EEp=EEp1,
                            CH=24576)
    X2 = out2.reshape(CN, HEADS, EMB).mean(axis=1) + b2

    routes9 = routes[..., :RSTEPS].reshape(-1)
    rt = X2[routes9].reshape(B * L, RSTEPS, EMB)
    idx = jnp.maximum(routes_len.reshape(-1) - 1, 0).astype(jnp.int32)
    wcat_r = jnp.concatenate([Wih_r.T, Whh_r.T], axis=0)
    br = (bih_r + bhh_r)[None, :]
    xt = _route_lstm(rt, idx, wcat_r, br, blk=800)
    xt = xt.reshape(B, L, HID)

    qt = W[questions]
    r = result_emb[answers]
    xcat = jnp.concatenate([qt, xt, r], axis=-1)
    ind = attn0[questions[:, 1:]]

    state, pred = _kt_chain(
        xcat, fc0_w, fc0_b[None, :], Wih_k.T, Whh_k.T,
        (bih_k + bhh_k)[None, :], fc1_w, fc1_b[None, :], ind)
    return (attn0, state, pred)
